# baseline (speedup 1.0000x reference)
import functools

import jax
import jax.numpy as jnp
from jax import lax
from jax.experimental import pallas as pl
from jax.experimental.pallas import tpu as pltpu
from jax.experimental.pallas import tpu_sc as plsc

N = 10000
ROWS = 80
NP = ROWS * 128
E = 320000
NC, NS = 2, 16
NW = NC * NS
EPW = E // NW
GPW = EPW // 16
RPT = 8
NWRITERS = ROWS // RPT
NGROUPS = ROWS * 8
REAL_GROUPS = N // 16

_f32 = jnp.float32
_i32 = jnp.int32


def _mesh():
    return plsc.VectorSubcoreMesh(
        core_axis_name="c", subcore_axis_name="s",
        num_cores=NC, num_subcores=NS)


_SC_PARAMS = pltpu.CompilerParams(needs_layout_passes=False)


def _g_slices(g):
    return g >> 3, (g & 7) * 16


def _zero_groups(ref, lo, hi):
    z = jnp.zeros((16,), _f32)

    def bd(r, carry):
        for k in range(8):
            ref[r, pl.ds(k * 16, 16)] = z
        return carry

    lax.fori_loop(lo // 8, hi // 8, bd, 0)


def _fill_rowids(rowid_ref):
    for k in range(ROWS // 16):
        rowid_ref[pl.ds(k * 16, 16)] = lax.iota(_i32, 16) + k * 16


EDGE_UNROLL = 5


def _edge_scatter(src_v, vals_fn, acc_v):
    def bd(i, carry):
        for k in range(EDGE_UNROLL):
            g = i * EDGE_UNROLL + k
            si = src_v[pl.ds(g * 16, 16)]
            vals = vals_fn(g)
            plsc.addupdate_scatter(acc_v, [si >> 7, si & 127], vals)
        return carry

    lax.fori_loop(0, GPW // EDGE_UNROLL, bd, 0)


def _prologue(sid, acc_v, shared):
    _zero_groups(acc_v, 0, NGROUPS)

    @pl.when(sid == 0)
    def _():
        pltpu.sync_copy(acc_v, shared)

    plsc.subcore_barrier()


def _combine_and_emit(sid, cid, acc_v, rowid_ref, shared, parts_ref):
    pltpu.sync_copy(acc_v, shared.at[rowid_ref], add=True)
    plsc.subcore_barrier()

    @pl.when(sid < NWRITERS)
    def _():
        row0 = sid * RPT
        pltpu.sync_copy(shared.at[pl.ds(row0, RPT)],
                        parts_ref.at[cid, pl.ds(row0, RPT)])


@functools.partial(
    pl.kernel,
    out_type=jax.ShapeDtypeStruct((NC, ROWS, 128), _f32),
    mesh=_mesh(),
    compiler_params=_SC_PARAMS,
    scratch_types=[
        pltpu.VMEM((EPW,), _i32),
        pltpu.VMEM((ROWS, 128), _f32),
        pltpu.VMEM((ROWS,), _i32),
        pltpu.VMEM_SHARED((ROWS, 128), _f32),
    ],
)
def _sc_degree(dst_hbm, parts_ref, dst_v, acc_v, rowid_ref, shared):
    cid = lax.axis_index("c")
    sid = lax.axis_index("s")
    wid = sid * NC + cid
    pltpu.sync_copy(dst_hbm.at[pl.ds(wid * EPW, EPW)], dst_v)
    _fill_rowids(rowid_ref)
    _prologue(sid, acc_v, shared)
    ones = jnp.ones((16,), _f32)
    _edge_scatter(dst_v, lambda g: ones, acc_v)
    _combine_and_emit(sid, cid, acc_v, rowid_ref, shared, parts_ref)


def _make_round(mode):
    scratch = [
        pltpu.VMEM((EPW,), _i32),
        pltpu.VMEM((EPW,), _i32),
        pltpu.VMEM((ROWS, 128), _f32),
        pltpu.VMEM((ROWS, 128), _f32),
        pltpu.VMEM((ROWS, 128), _f32),
        pltpu.VMEM((ROWS, 128), _f32),
        pltpu.VMEM((ROWS, 128), _f32),
        pltpu.VMEM((ROWS,), _i32),
        pltpu.VMEM_SHARED((ROWS, 128), _f32),
    ]
    out_node = jax.ShapeDtypeStruct((ROWS, 128), _f32)
    out_parts = jax.ShapeDtypeStruct((NC, ROWS, 128), _f32)

    def body(src_hbm, dst_hbm, w_hbm, base_hbm, parts_hbm, node_out,
             parts_out, dst_v, src_v, ta, tb, w_v, p_v, acc_v, rowid_ref,
             shared):
        cid = lax.axis_index("c")
        sid = lax.axis_index("s")
        wid = sid * NC + cid
        row0 = sid * RPT
        pltpu.sync_copy(dst_hbm.at[pl.ds(wid * EPW, EPW)], dst_v)
        pltpu.sync_copy(src_hbm.at[pl.ds(wid * EPW, EPW)], src_v)
        pltpu.sync_copy(parts_hbm.at[0], ta)
        pltpu.sync_copy(parts_hbm.at[1], tb)
        _fill_rowids(rowid_ref)

        if mode == 1:
            def wb(r, carry):
                for k in range(8):
                    c = k * 16
                    dv = ta[r, pl.ds(c, 16)] + tb[r, pl.ds(c, 16)]
                    p_v[r, pl.ds(c, 16)] = 1.0 / jnp.maximum(dv, 1.0)
                return carry

            lax.fori_loop(0, ROWS, wb, 0)
        else:
            pltpu.sync_copy(w_hbm, w_v)
            if mode == 3:
                pltpu.sync_copy(base_hbm, acc_v)

            def ub_one(r, c):
                tv = ta[r, pl.ds(c, 16)] + tb[r, pl.ds(c, 16)]
                if mode == 2:
                    uv = 1.0 + tv
                else:
                    uv = acc_v[r, pl.ds(c, 16)] + tv
                ta[r, pl.ds(c, 16)] = uv
                p_v[r, pl.ds(c, 16)] = uv * w_v[r, pl.ds(c, 16)]

            def ub(r, carry):
                for k in range(8):
                    ub_one(r, k * 16)
                return carry

            lax.fori_loop(0, (REAL_GROUPS - 1) // 8, ub, 0)
            ub_one((REAL_GROUPS - 1) // 8, 0)
            z16 = jnp.zeros((16,), _f32)
            for k in range(1, 8):
                ta[(REAL_GROUPS - 1) // 8, pl.ds(k * 16, 16)] = z16
                p_v[(REAL_GROUPS - 1) // 8, pl.ds(k * 16, 16)] = z16
            _zero_groups(ta, NGROUPS - 8, NGROUPS)
            _zero_groups(p_v, NGROUPS - 8, NGROUPS)

        emit_src = p_v if mode == 1 else ta

        @pl.when((cid == 0) & (sid < NWRITERS))
        def _():
            pltpu.sync_copy(emit_src.at[pl.ds(row0, RPT)],
                            node_out.at[pl.ds(row0, RPT)])

        _prologue(sid, acc_v, shared)

        def vals_fn(g):
            di = dst_v[pl.ds(g * 16, 16)]
            return plsc.load_gather(p_v, [di >> 7, di & 127])

        _edge_scatter(src_v, vals_fn, acc_v)
        _combine_and_emit(sid, cid, acc_v, rowid_ref, shared, parts_out)

    if mode == 1:
        @functools.partial(
            pl.kernel, out_type=(out_node, out_parts), mesh=_mesh(),
            compiler_params=_SC_PARAMS, scratch_types=scratch)
        def round_fn(src_hbm, dst_hbm, parts_hbm, node_out, parts_out,
                     *scratch_refs):
            body(src_hbm, dst_hbm, None, None, parts_hbm, node_out,
                 parts_out, *scratch_refs)
    elif mode == 2:
        @functools.partial(
            pl.kernel, out_type=(out_node, out_parts), mesh=_mesh(),
            compiler_params=_SC_PARAMS, scratch_types=scratch)
        def round_fn(src_hbm, dst_hbm, w_hbm, parts_hbm, node_out,
                     parts_out, *scratch_refs):
            body(src_hbm, dst_hbm, w_hbm, None, parts_hbm, node_out,
                 parts_out, *scratch_refs)
    else:
        @functools.partial(
            pl.kernel, out_type=(out_node, out_parts), mesh=_mesh(),
            compiler_params=_SC_PARAMS, scratch_types=scratch)
        def round_fn(src_hbm, dst_hbm, w_hbm, base_hbm, parts_hbm,
                     node_out, parts_out, *scratch_refs):
            body(src_hbm, dst_hbm, w_hbm, base_hbm, parts_hbm, node_out,
                 parts_out, *scratch_refs)
    return round_fn


_sc_round1 = _make_round(1)
_sc_round2 = _make_round(2)
_sc_round3 = _make_round(3)


def _assemble_body(x_ref, u1_ref, u2_ref, t3a_ref, t3b_ref, wp_ref, bp_ref,
                   wl_ref, bl_ref, wo_ref, bo_ref, out_ref):
    u1 = u1_ref[...]
    u2 = u2_ref[...]
    u3 = u2 + t3a_ref[...] + t3b_ref[...]
    U = jnp.concatenate([u1, u2, u3], axis=0)
    V = jnp.dot(U, x_ref[...], preferred_element_type=_f32, precision=lax.Precision.HIGHEST)
    s1, s2, s3 = jnp.sum(u1), jnp.sum(u2), jnp.sum(u3)
    bp = bp_ref[...]
    bl = bl_ref[...]
    wl = wl_ref[...]
    A = jnp.dot(V, wp_ref[...], preferred_element_type=_f32, precision=lax.Precision.HIGHEST)
    a1 = A[0:1] + s1 * bp
    a2 = A[1:2] + s2 * bp
    a3 = A[2:3] + s3 * bp
    nf = jnp.float32(N)
    g1 = jnp.dot(a1, wl, preferred_element_type=_f32, precision=lax.Precision.HIGHEST) + nf * bl
    q = jnp.dot(a2, wl, preferred_element_type=_f32, precision=lax.Precision.HIGHEST) + s1 * bl
    g2 = jnp.dot(q, wl, preferred_element_type=_f32, precision=lax.Precision.HIGHEST) + nf * bl
    r = jnp.dot(a3, wl, preferred_element_type=_f32, precision=lax.Precision.HIGHEST) + s2 * bl
    q2 = jnp.dot(r, wl, preferred_element_type=_f32, precision=lax.Precision.HIGHEST) + s1 * bl
    g3 = jnp.dot(q2, wl, preferred_element_type=_f32, precision=lax.Precision.HIGHEST) + nf * bl
    cat = jnp.concatenate([g1, g2, g3], axis=1)
    out = jnp.dot(cat, wo_ref[...], preferred_element_type=_f32, precision=lax.Precision.HIGHEST) / nf
    out_ref[...] = out + bo_ref[...]


def _assemble(x_pad, u1r, u2r, t3ar, t3br, wp, bp, wl, bl, wo, bo):
    return pl.pallas_call(
        _assemble_body,
        out_shape=jax.ShapeDtypeStruct((1, 128), _f32),
    )(x_pad, u1r, u2r, t3ar, t3br, wp, bp, wl, bl, wo, bo)


def kernel(x, edge_index, W_proj, b_proj, W_lin, b_lin, W_out, b_out):
    src = edge_index[0]
    dst = edge_index[1]

    deg_parts = _sc_degree(dst)
    w_nodes, t1_parts = _sc_round1(src, dst, deg_parts)
    u1_nodes, t2_parts = _sc_round2(src, dst, w_nodes, t1_parts)
    u2_nodes, t3_parts = _sc_round3(src, dst, w_nodes, u1_nodes, t2_parts)

    x_pad = jnp.concatenate(
        [x, jnp.zeros((NP - N, x.shape[1]), _f32)], axis=0)
    u1r = u1_nodes.reshape(1, NP)
    u2r = u2_nodes.reshape(1, NP)
    t3ar = t3_parts[0].reshape(1, NP)
    t3br = t3_parts[1].reshape(1, NP)
    out = _assemble(x_pad, u1r, u2r, t3ar, t3br,
                    W_proj, b_proj.reshape(1, 128), W_lin,
                    b_lin.reshape(1, 128), W_out, b_out.reshape(1, 128))
    return out.reshape(128)


# speedup vs baseline: 1.1605x; 1.1605x over previous
import functools

import jax
import jax.numpy as jnp
from jax import lax
from jax.experimental import pallas as pl
from jax.experimental.pallas import tpu as pltpu
from jax.experimental.pallas import tpu_sc as plsc

N = 10000
ROWS = 80
NP = ROWS * 128
E = 320000
NC, NS = 2, 16
NW = NC * NS
EPW = E // NW
GPW = EPW // 16
RPT = 8
NWRITERS = ROWS // RPT
NGROUPS = ROWS * 8
REAL_GROUPS = N // 16

_f32 = jnp.float32
_i32 = jnp.int32


def _mesh():
    return plsc.VectorSubcoreMesh(
        core_axis_name="c", subcore_axis_name="s",
        num_cores=NC, num_subcores=NS)


_SC_PARAMS = pltpu.CompilerParams(needs_layout_passes=False)


def _g_slices(g):
    return g >> 3, (g & 7) * 16


def _zero_groups(ref, lo, hi):
    z = jnp.zeros((16,), _f32)

    def bd(r, carry):
        for k in range(8):
            ref[r, pl.ds(k * 16, 16)] = z
        return carry

    lax.fori_loop(lo // 8, hi // 8, bd, 0)


def _fill_rowids(rowid_ref):
    for k in range(ROWS // 16):
        rowid_ref[pl.ds(k * 16, 16)] = lax.iota(_i32, 16) + k * 16


EDGE_UNROLL = 5


def _edge_scatter(src_v, vals_fn, acc_v):
    @plsc.parallel_loop(0, GPW, unroll=EDGE_UNROLL)
    def _(g):
        si = src_v[pl.ds(g * 16, 16)]
        vals = vals_fn(g)
        plsc.addupdate_scatter(acc_v, [si >> 7, si & 127], vals)


def _prologue(sid, acc_v, shared):
    _zero_groups(acc_v, 0, NGROUPS)

    @pl.when(sid == 0)
    def _():
        pltpu.sync_copy(acc_v, shared)

    plsc.subcore_barrier()


def _combine_and_emit(sid, cid, acc_v, rowid_ref, shared, parts_ref):
    pltpu.sync_copy(acc_v, shared.at[rowid_ref], add=True)
    plsc.subcore_barrier()

    @pl.when(sid < NWRITERS)
    def _():
        row0 = sid * RPT
        pltpu.sync_copy(shared.at[pl.ds(row0, RPT)],
                        parts_ref.at[cid, pl.ds(row0, RPT)])


@functools.partial(
    pl.kernel,
    out_type=jax.ShapeDtypeStruct((NC, ROWS, 128), _f32),
    mesh=_mesh(),
    compiler_params=_SC_PARAMS,
    scratch_types=[
        pltpu.VMEM((EPW,), _i32),
        pltpu.VMEM((ROWS, 128), _f32),
        pltpu.VMEM((ROWS,), _i32),
        pltpu.VMEM_SHARED((ROWS, 128), _f32),
    ],
)
def _sc_degree(dst_hbm, parts_ref, dst_v, acc_v, rowid_ref, shared):
    cid = lax.axis_index("c")
    sid = lax.axis_index("s")
    wid = sid * NC + cid
    pltpu.sync_copy(dst_hbm.at[pl.ds(wid * EPW, EPW)], dst_v)
    _fill_rowids(rowid_ref)
    _prologue(sid, acc_v, shared)
    ones = jnp.ones((16,), _f32)
    _edge_scatter(dst_v, lambda g: ones, acc_v)
    _combine_and_emit(sid, cid, acc_v, rowid_ref, shared, parts_ref)


def _make_round(mode):
    scratch = [
        pltpu.VMEM((EPW,), _i32),
        pltpu.VMEM((EPW,), _i32),
        pltpu.VMEM((ROWS, 128), _f32),
        pltpu.VMEM((ROWS, 128), _f32),
        pltpu.VMEM((ROWS, 128), _f32),
        pltpu.VMEM((ROWS, 128), _f32),
        pltpu.VMEM((ROWS, 128), _f32),
        pltpu.VMEM((ROWS,), _i32),
        pltpu.VMEM_SHARED((ROWS, 128), _f32),
    ]
    out_node = jax.ShapeDtypeStruct((ROWS, 128), _f32)
    out_parts = jax.ShapeDtypeStruct((NC, ROWS, 128), _f32)

    def body(src_hbm, dst_hbm, w_hbm, base_hbm, parts_hbm, node_out,
             parts_out, dst_v, src_v, ta, tb, w_v, p_v, acc_v, rowid_ref,
             shared):
        cid = lax.axis_index("c")
        sid = lax.axis_index("s")
        wid = sid * NC + cid
        row0 = sid * RPT
        pltpu.sync_copy(dst_hbm.at[pl.ds(wid * EPW, EPW)], dst_v)
        pltpu.sync_copy(src_hbm.at[pl.ds(wid * EPW, EPW)], src_v)
        pltpu.sync_copy(parts_hbm.at[0], ta)
        pltpu.sync_copy(parts_hbm.at[1], tb)
        _fill_rowids(rowid_ref)

        if mode == 1:
            def wb(r, carry):
                for k in range(8):
                    c = k * 16
                    dv = ta[r, pl.ds(c, 16)] + tb[r, pl.ds(c, 16)]
                    p_v[r, pl.ds(c, 16)] = 1.0 / jnp.maximum(dv, 1.0)
                return carry

            lax.fori_loop(0, ROWS, wb, 0)
        else:
            pltpu.sync_copy(w_hbm, w_v)
            if mode == 3:
                pltpu.sync_copy(base_hbm, acc_v)

            def ub_one(r, c):
                tv = ta[r, pl.ds(c, 16)] + tb[r, pl.ds(c, 16)]
                if mode == 2:
                    uv = 1.0 + tv
                else:
                    uv = acc_v[r, pl.ds(c, 16)] + tv
                ta[r, pl.ds(c, 16)] = uv
                p_v[r, pl.ds(c, 16)] = uv * w_v[r, pl.ds(c, 16)]

            def ub(r, carry):
                for k in range(8):
                    ub_one(r, k * 16)
                return carry

            lax.fori_loop(0, (REAL_GROUPS - 1) // 8, ub, 0)
            ub_one((REAL_GROUPS - 1) // 8, 0)
            z16 = jnp.zeros((16,), _f32)
            for k in range(1, 8):
                ta[(REAL_GROUPS - 1) // 8, pl.ds(k * 16, 16)] = z16
                p_v[(REAL_GROUPS - 1) // 8, pl.ds(k * 16, 16)] = z16
            _zero_groups(ta, NGROUPS - 8, NGROUPS)
            _zero_groups(p_v, NGROUPS - 8, NGROUPS)

        emit_src = p_v if mode == 1 else ta

        @pl.when((cid == 0) & (sid < NWRITERS))
        def _():
            pltpu.sync_copy(emit_src.at[pl.ds(row0, RPT)],
                            node_out.at[pl.ds(row0, RPT)])

        _prologue(sid, acc_v, shared)

        def vals_fn(g):
            di = dst_v[pl.ds(g * 16, 16)]
            return plsc.load_gather(p_v, [di >> 7, di & 127])

        _edge_scatter(src_v, vals_fn, acc_v)
        _combine_and_emit(sid, cid, acc_v, rowid_ref, shared, parts_out)

    if mode == 1:
        @functools.partial(
            pl.kernel, out_type=(out_node, out_parts), mesh=_mesh(),
            compiler_params=_SC_PARAMS, scratch_types=scratch)
        def round_fn(src_hbm, dst_hbm, parts_hbm, node_out, parts_out,
                     *scratch_refs):
            body(src_hbm, dst_hbm, None, None, parts_hbm, node_out,
                 parts_out, *scratch_refs)
    elif mode == 2:
        @functools.partial(
            pl.kernel, out_type=(out_node, out_parts), mesh=_mesh(),
            compiler_params=_SC_PARAMS, scratch_types=scratch)
        def round_fn(src_hbm, dst_hbm, w_hbm, parts_hbm, node_out,
                     parts_out, *scratch_refs):
            body(src_hbm, dst_hbm, w_hbm, None, parts_hbm, node_out,
                 parts_out, *scratch_refs)
    else:
        @functools.partial(
            pl.kernel, out_type=(out_node, out_parts), mesh=_mesh(),
            compiler_params=_SC_PARAMS, scratch_types=scratch)
        def round_fn(src_hbm, dst_hbm, w_hbm, base_hbm, parts_hbm,
                     node_out, parts_out, *scratch_refs):
            body(src_hbm, dst_hbm, w_hbm, base_hbm, parts_hbm, node_out,
                 parts_out, *scratch_refs)
    return round_fn


_sc_round1 = _make_round(1)
_sc_round2 = _make_round(2)
_sc_round3 = _make_round(3)


def _assemble_body(x_ref, u1_ref, u2_ref, t3a_ref, t3b_ref, wp_ref, bp_ref,
                   wl_ref, bl_ref, wo_ref, bo_ref, out_ref):
    u1 = u1_ref[...]
    u2 = u2_ref[...]
    u3 = u2 + t3a_ref[...] + t3b_ref[...]
    U = jnp.concatenate([u1, u2, u3], axis=0)
    V = jnp.dot(U, x_ref[...], preferred_element_type=_f32, precision=lax.Precision.HIGHEST)
    s1, s2, s3 = jnp.sum(u1), jnp.sum(u2), jnp.sum(u3)
    bp = bp_ref[...]
    bl = bl_ref[...]
    wl = wl_ref[...]
    A = jnp.dot(V, wp_ref[...], preferred_element_type=_f32, precision=lax.Precision.HIGHEST)
    a1 = A[0:1] + s1 * bp
    a2 = A[1:2] + s2 * bp
    a3 = A[2:3] + s3 * bp
    nf = jnp.float32(N)
    g1 = jnp.dot(a1, wl, preferred_element_type=_f32, precision=lax.Precision.HIGHEST) + nf * bl
    q = jnp.dot(a2, wl, preferred_element_type=_f32, precision=lax.Precision.HIGHEST) + s1 * bl
    g2 = jnp.dot(q, wl, preferred_element_type=_f32, precision=lax.Precision.HIGHEST) + nf * bl
    r = jnp.dot(a3, wl, preferred_element_type=_f32, precision=lax.Precision.HIGHEST) + s2 * bl
    q2 = jnp.dot(r, wl, preferred_element_type=_f32, precision=lax.Precision.HIGHEST) + s1 * bl
    g3 = jnp.dot(q2, wl, preferred_element_type=_f32, precision=lax.Precision.HIGHEST) + nf * bl
    cat = jnp.concatenate([g1, g2, g3], axis=1)
    out = jnp.dot(cat, wo_ref[...], preferred_element_type=_f32, precision=lax.Precision.HIGHEST) / nf
    out_ref[...] = out + bo_ref[...]


def _assemble(x_pad, u1r, u2r, t3ar, t3br, wp, bp, wl, bl, wo, bo):
    return pl.pallas_call(
        _assemble_body,
        out_shape=jax.ShapeDtypeStruct((1, 128), _f32),
    )(x_pad, u1r, u2r, t3ar, t3br, wp, bp, wl, bl, wo, bo)


def kernel(x, edge_index, W_proj, b_proj, W_lin, b_lin, W_out, b_out):
    src = edge_index[0]
    dst = edge_index[1]

    deg_parts = _sc_degree(dst)
    w_nodes, t1_parts = _sc_round1(src, dst, deg_parts)
    u1_nodes, t2_parts = _sc_round2(src, dst, w_nodes, t1_parts)
    u2_nodes, t3_parts = _sc_round3(src, dst, w_nodes, u1_nodes, t2_parts)

    x_pad = jnp.concatenate(
        [x, jnp.zeros((NP - N, x.shape[1]), _f32)], axis=0)
    u1r = u1_nodes.reshape(1, NP)
    u2r = u2_nodes.reshape(1, NP)
    t3ar = t3_parts[0].reshape(1, NP)
    t3br = t3_parts[1].reshape(1, NP)
    out = _assemble(x_pad, u1r, u2r, t3ar, t3br,
                    W_proj, b_proj.reshape(1, 128), W_lin,
                    b_lin.reshape(1, 128), W_out, b_out.reshape(1, 128))
    return out.reshape(128)
